# Initial kernel scaffold; baseline (speedup 1.0000x reference)
import functools

import jax
import jax.numpy as jnp
from jax import lax
from jax.experimental import pallas as pl
from jax.experimental.pallas import tpu as pltpu
from jax.experimental.pallas import tpu_sc as plsc

N = 10000
E = 320000
F_IN = 128
H = 64
L = 3
C = 10
G = 64
BN_EPS = 1e-5

NC = 2
NS = 16
CHUNK = 80
NCHUNK = E // (NC * NS * CHUNK)
ROWS_PER_TILE = N // NS
ZROWS = 25


def _segsum_body(h_hbm, src_hbm, dst_hbm, out_hbm,
                 sidx, didx, gbuf, zbuf, acc, sem):
    c = lax.axis_index("c")
    s = lax.axis_index("s")
    wid = c * NS + s

    for r in range(ZROWS):
        for q in range(H // 16):
            zbuf[r, pl.ds(q * 16, 16)] = jnp.zeros((16,), jnp.float32)
    base = s * ROWS_PER_TILE

    def zloop(k, carry):
        pltpu.sync_copy(zbuf, acc.at[pl.ds(base + k * ZROWS, ZROWS)])
        return carry

    lax.fori_loop(0, ROWS_PER_TILE // ZROWS, zloop, 0)

    row0 = wid * NCHUNK
    pltpu.sync_copy(src_hbm.at[pl.ds(row0, NCHUNK)], sidx)
    pltpu.sync_copy(dst_hbm.at[pl.ds(row0, NCHUNK)], didx)

    plsc.subcore_barrier()

    def eloop(j, carry):
        pltpu.async_copy(h_hbm.at[sidx.at[j]], gbuf, sem).wait()
        pltpu.sync_copy(gbuf, acc.at[didx.at[j]], add=True)
        return carry

    lax.fori_loop(0, NCHUNK, eloop, 0)

    plsc.subcore_barrier()
    pltpu.sync_copy(acc.at[pl.ds(base, ROWS_PER_TILE)],
                    out_hbm.at[c, pl.ds(base, ROWS_PER_TILE)])


_segsum_call = pl.kernel(
    _segsum_body,
    out_type=jax.ShapeDtypeStruct((NC, N, H), jnp.float32),
    mesh=plsc.VectorSubcoreMesh(core_axis_name="c", subcore_axis_name="s",
                                num_cores=NC, num_subcores=NS),
    scratch_types=[
        pltpu.VMEM((NCHUNK, CHUNK), jnp.int32),
        pltpu.VMEM((NCHUNK, CHUNK), jnp.int32),
        pltpu.VMEM((CHUNK, H), jnp.float32),
        pltpu.VMEM((ZROWS, H), jnp.float32),
        pltpu.VMEM_SHARED((N, H), jnp.float32),
        pltpu.SemaphoreType.DMA,
    ],
    name="gin_segsum_sc",
)


def _enc_body(x_ref, w_ref, b_ref, out_ref):
    out_ref[:, :] = jnp.dot(x_ref[:, :], w_ref[:, :],
                            preferred_element_type=jnp.float32) + b_ref[:, :]


_enc_call = pl.pallas_call(
    _enc_body,
    out_shape=jax.ShapeDtypeStruct((N, H), jnp.float32),
    name="gin_encoder_tc",
)


def _mlp_body(h_ref, p0_ref, p1_ref, w1_ref, b1_ref, w2_ref, b2_ref,
              gam_ref, bet_ref, eps_ref, out_ref):
    a = (1.0 + eps_ref[0, 0]) * h_ref[:, :] + p0_ref[:, :] + p1_ref[:, :]
    t = jnp.dot(a, w1_ref[:, :], preferred_element_type=jnp.float32)
    t = jnp.maximum(t + b1_ref[:, :], 0.0)
    h2 = jnp.dot(t, w2_ref[:, :], preferred_element_type=jnp.float32)
    h2 = h2 + b2_ref[:, :]
    mean = jnp.mean(h2, axis=0, keepdims=True)
    var = jnp.mean((h2 - mean) ** 2, axis=0, keepdims=True)
    hn = (h2 - mean) / jnp.sqrt(var + BN_EPS) * gam_ref[:, :] + bet_ref[:, :]
    out_ref[:, :] = jnp.maximum(hn, 0.0)


_mlp_call = pl.pallas_call(
    _mlp_body,
    out_shape=jax.ShapeDtypeStruct((N, H), jnp.float32),
    name="gin_layer_tc",
)


def _pool_body(h_ref, batch_ref, w_ref, b_ref, out_ref):
    gids = lax.broadcasted_iota(jnp.int32, (G, 1), 0)
    mask = (batch_ref[:, :] == gids).astype(jnp.float32)
    sums = jnp.dot(mask, h_ref[:, :], preferred_element_type=jnp.float32)
    counts = jnp.sum(mask, axis=1, keepdims=True)
    pooled = sums / jnp.maximum(counts, 1.0)
    out_ref[:, :] = jnp.dot(pooled, w_ref[:, :],
                            preferred_element_type=jnp.float32) + b_ref[:, :]


_pool_call = pl.pallas_call(
    _pool_body,
    out_shape=jax.ShapeDtypeStruct((G, C), jnp.float32),
    name="gin_pool_tc",
)


def kernel(x, edge_index, batch, enc_W, enc_b, eps, W1, b1, W2, b2,
           gamma, beta, lin_W, lin_b):
    src2 = edge_index[0].reshape(E // CHUNK, CHUNK)
    dst2 = edge_index[1].reshape(E // CHUNK, CHUNK)
    h = _enc_call(x, enc_W, enc_b.reshape(1, H))
    for i in range(L):
        parts = _segsum_call(h, src2, dst2)
        h = _mlp_call(h, parts[0], parts[1], W1[i], b1[i].reshape(1, H),
                      W2[i], b2[i].reshape(1, H), gamma[i].reshape(1, H),
                      beta[i].reshape(1, H), eps[i].reshape(1, 1))
    return _pool_call(h, batch.reshape(1, N), lin_W, lin_b.reshape(1, C))


# speedup vs baseline: 7.4510x; 7.4510x over previous
import functools

import jax
import jax.numpy as jnp
from jax import lax
from jax.experimental import pallas as pl
from jax.experimental.pallas import tpu as pltpu
from jax.experimental.pallas import tpu_sc as plsc

N = 10000
E = 320000
F_IN = 128
H = 64
L = 3
C = 10
G = 64
BN_EPS = 1e-5

NC = 2
NS = 16
NW = NC * NS
CHUNK = 80
NCHUNK = E // (NW * CHUNK)
N_PAD = 10240
ROWS_PER_TILE = N_PAD // NS
ZROWS = 32


def _segsum_body(h_hbm, src_hbm, dst_hbm, out_hbm,
                 sidx, didx, gbuf, zbuf, acc, sem):
    c = lax.axis_index("c")
    s = lax.axis_index("s")
    wid = c * NS + s

    for r in range(ZROWS):
        for q in range(H // 16):
            zbuf[r, pl.ds(q * 16, 16)] = jnp.zeros((16,), jnp.float32)
    base = s * ROWS_PER_TILE

    def zloop(k, carry):
        pltpu.sync_copy(zbuf, acc.at[pl.ds(base + k * ZROWS, ZROWS)])
        return carry

    lax.fori_loop(0, ROWS_PER_TILE // ZROWS, zloop, 0)

    pltpu.sync_copy(src_hbm.at[wid], sidx)
    pltpu.sync_copy(dst_hbm.at[wid], didx)

    plsc.subcore_barrier()

    def eloop(j, carry):
        pltpu.async_copy(h_hbm.at[sidx.at[j]], gbuf, sem).wait()
        pltpu.sync_copy(gbuf, acc.at[didx.at[j]], add=True)
        return carry

    lax.fori_loop(0, NCHUNK, eloop, 0)

    plsc.subcore_barrier()
    pltpu.sync_copy(acc.at[pl.ds(base, ROWS_PER_TILE)],
                    out_hbm.at[c, pl.ds(base, ROWS_PER_TILE)])


_segsum_call = pl.kernel(
    _segsum_body,
    out_type=jax.ShapeDtypeStruct((NC, N_PAD, H), jnp.float32),
    mesh=plsc.VectorSubcoreMesh(core_axis_name="c", subcore_axis_name="s",
                                num_cores=NC, num_subcores=NS),
    scratch_types=[
        pltpu.VMEM((NCHUNK, CHUNK), jnp.int32),
        pltpu.VMEM((NCHUNK, CHUNK), jnp.int32),
        pltpu.VMEM((CHUNK, H), jnp.float32),
        pltpu.VMEM((ZROWS, H), jnp.float32),
        pltpu.VMEM_SHARED((N_PAD, H), jnp.float32),
        pltpu.SemaphoreType.DMA,
    ],
    compiler_params=pltpu.CompilerParams(use_tc_tiling_on_sc=False),
    name="gin_segsum_sc",
)


def _enc_body(x_ref, w_ref, b_ref, out_ref):
    out_ref[:, :] = jnp.dot(x_ref[:, :], w_ref[:, :],
                            preferred_element_type=jnp.float32) + b_ref[:, :]


_enc_call = pl.pallas_call(
    _enc_body,
    out_shape=jax.ShapeDtypeStruct((N, H), jnp.float32),
    name="gin_encoder_tc",
)


def _mlp_body(h_ref, p0_ref, p1_ref, w1_ref, b1_ref, w2_ref, b2_ref,
              gam_ref, bet_ref, eps_ref, out_ref):
    a = (1.0 + eps_ref[0, 0]) * h_ref[:, :] + p0_ref[:N, :] + p1_ref[:N, :]
    t = jnp.dot(a, w1_ref[:, :], preferred_element_type=jnp.float32)
    t = jnp.maximum(t + b1_ref[:, :], 0.0)
    h2 = jnp.dot(t, w2_ref[:, :], preferred_element_type=jnp.float32)
    h2 = h2 + b2_ref[:, :]
    mean = jnp.mean(h2, axis=0, keepdims=True)
    var = jnp.mean((h2 - mean) ** 2, axis=0, keepdims=True)
    hn = (h2 - mean) / jnp.sqrt(var + BN_EPS) * gam_ref[:, :] + bet_ref[:, :]
    out_ref[:, :] = jnp.maximum(hn, 0.0)


_mlp_call = pl.pallas_call(
    _mlp_body,
    out_shape=jax.ShapeDtypeStruct((N, H), jnp.float32),
    name="gin_layer_tc",
)


def _pool_body(h_ref, batch_ref, w_ref, b_ref, out_ref):
    gids = lax.broadcasted_iota(jnp.int32, (G, 1), 0)
    mask = (batch_ref[:, :] == gids).astype(jnp.float32)
    sums = jnp.dot(mask, h_ref[:, :], preferred_element_type=jnp.float32)
    counts = jnp.sum(mask, axis=1, keepdims=True)
    pooled = sums / jnp.maximum(counts, 1.0)
    out_ref[:, :] = jnp.dot(pooled, w_ref[:, :],
                            preferred_element_type=jnp.float32) + b_ref[:, :]


_pool_call = pl.pallas_call(
    _pool_body,
    out_shape=jax.ShapeDtypeStruct((G, C), jnp.float32),
    name="gin_pool_tc",
)


def kernel(x, edge_index, batch, enc_W, enc_b, eps, W1, b1, W2, b2,
           gamma, beta, lin_W, lin_b):
    src2 = edge_index[0].reshape(NW, NCHUNK, CHUNK)
    dst2 = edge_index[1].reshape(NW, NCHUNK, CHUNK)
    h = _enc_call(x, enc_W, enc_b.reshape(1, H))
    for i in range(L):
        parts = _segsum_call(h, src2, dst2)
        h = _mlp_call(h, parts[0], parts[1], W1[i], b1[i].reshape(1, H),
                      W2[i], b2[i].reshape(1, H), gamma[i].reshape(1, H),
                      beta[i].reshape(1, H), eps[i].reshape(1, 1))
    return _pool_call(h, batch.reshape(1, N), lin_W, lin_b.reshape(1, C))


# speedup vs baseline: 14.5408x; 1.9515x over previous
import functools

import jax
import jax.numpy as jnp
from jax import lax
from jax.experimental import pallas as pl
from jax.experimental.pallas import tpu as pltpu
from jax.experimental.pallas import tpu_sc as plsc

N = 10000
E = 320000
F_IN = 128
H = 64
L = 3
C = 10
G = 64
BN_EPS = 1e-5

NC = 2
NS = 16
NW = NC * NS
CHUNK = 80
NCHUNK = E // (NW * CHUNK)
N_PAD = 10240
ROWS_PER_TILE = N_PAD // NS
ZROWS = 32
NBUF = 5


def _segsum_body(h_hbm, src_hbm, dst_hbm, out_hbm,
                 sidx, didx, gbuf, zbuf, acc, *gsem):
    c = lax.axis_index("c")
    s = lax.axis_index("s")
    wid = c * NS + s

    for r in range(ZROWS):
        for q in range(H // 16):
            zbuf[r, pl.ds(q * 16, 16)] = jnp.zeros((16,), jnp.float32)
    base = s * ROWS_PER_TILE

    def zloop(k, carry):
        pltpu.sync_copy(zbuf, acc.at[pl.ds(base + k * ZROWS, ZROWS)])
        return carry

    lax.fori_loop(0, ROWS_PER_TILE // ZROWS, zloop, 0)

    pltpu.sync_copy(src_hbm.at[wid], sidx)
    pltpu.sync_copy(dst_hbm.at[wid], didx)

    plsc.subcore_barrier()

    for b in range(NBUF):
        pltpu.async_copy(h_hbm.at[sidx.at[b]], gbuf.at[b], gsem[b])

    def group(io, carry):
        jo = io * NBUF
        for b in range(NBUF):
            j = jo + b
            pltpu.make_async_copy(h_hbm.at[sidx.at[b]],
                                  gbuf.at[b], gsem[b]).wait()
            pltpu.sync_copy(gbuf.at[b], acc.at[didx.at[j]], add=True)
            pltpu.async_copy(h_hbm.at[sidx.at[j + NBUF]], gbuf.at[b],
                             gsem[b])
        return carry

    lax.fori_loop(0, NCHUNK // NBUF - 1, group, 0)

    jo = NCHUNK - NBUF
    for b in range(NBUF):
        pltpu.make_async_copy(h_hbm.at[sidx.at[b]],
                              gbuf.at[b], gsem[b]).wait()
        pltpu.sync_copy(gbuf.at[b], acc.at[didx.at[jo + b]], add=True)

    plsc.subcore_barrier()
    pltpu.sync_copy(acc.at[pl.ds(base, ROWS_PER_TILE)],
                    out_hbm.at[c, pl.ds(base, ROWS_PER_TILE)])


_segsum_call = pl.kernel(
    _segsum_body,
    out_type=jax.ShapeDtypeStruct((NC, N_PAD, H), jnp.float32),
    mesh=plsc.VectorSubcoreMesh(core_axis_name="c", subcore_axis_name="s",
                                num_cores=NC, num_subcores=NS),
    scratch_types=[
        pltpu.VMEM((NCHUNK, CHUNK), jnp.int32),
        pltpu.VMEM((NCHUNK, CHUNK), jnp.int32),
        pltpu.VMEM((NBUF, CHUNK, H), jnp.float32),
        pltpu.VMEM((ZROWS, H), jnp.float32),
        pltpu.VMEM_SHARED((N_PAD, H), jnp.float32),
    ] + [pltpu.SemaphoreType.DMA] * NBUF,
    compiler_params=pltpu.CompilerParams(use_tc_tiling_on_sc=False),
    name="gin_segsum_sc",
)


def _enc_body(x_ref, w_ref, b_ref, out_ref):
    out_ref[:, :] = jnp.dot(x_ref[:, :], w_ref[:, :],
                            preferred_element_type=jnp.float32) + b_ref[:, :]


_enc_call = pl.pallas_call(
    _enc_body,
    out_shape=jax.ShapeDtypeStruct((N, H), jnp.float32),
    name="gin_encoder_tc",
)


def _mlp_body(h_ref, p0_ref, p1_ref, w1_ref, b1_ref, w2_ref, b2_ref,
              gam_ref, bet_ref, eps_ref, out_ref):
    a = (1.0 + eps_ref[0, 0]) * h_ref[:, :] + p0_ref[:N, :] + p1_ref[:N, :]
    t = jnp.dot(a, w1_ref[:, :], preferred_element_type=jnp.float32)
    t = jnp.maximum(t + b1_ref[:, :], 0.0)
    h2 = jnp.dot(t, w2_ref[:, :], preferred_element_type=jnp.float32)
    h2 = h2 + b2_ref[:, :]
    mean = jnp.mean(h2, axis=0, keepdims=True)
    var = jnp.mean((h2 - mean) ** 2, axis=0, keepdims=True)
    hn = (h2 - mean) / jnp.sqrt(var + BN_EPS) * gam_ref[:, :] + bet_ref[:, :]
    out_ref[:, :] = jnp.maximum(hn, 0.0)


_mlp_call = pl.pallas_call(
    _mlp_body,
    out_shape=jax.ShapeDtypeStruct((N, H), jnp.float32),
    name="gin_layer_tc",
)


def _pool_body(h_ref, batch_ref, w_ref, b_ref, out_ref):
    gids = lax.broadcasted_iota(jnp.int32, (G, 1), 0)
    mask = (batch_ref[:, :] == gids).astype(jnp.float32)
    sums = jnp.dot(mask, h_ref[:, :], preferred_element_type=jnp.float32)
    counts = jnp.sum(mask, axis=1, keepdims=True)
    pooled = sums / jnp.maximum(counts, 1.0)
    out_ref[:, :] = jnp.dot(pooled, w_ref[:, :],
                            preferred_element_type=jnp.float32) + b_ref[:, :]


_pool_call = pl.pallas_call(
    _pool_body,
    out_shape=jax.ShapeDtypeStruct((G, C), jnp.float32),
    name="gin_pool_tc",
)


def kernel(x, edge_index, batch, enc_W, enc_b, eps, W1, b1, W2, b2,
           gamma, beta, lin_W, lin_b):
    src2 = edge_index[0].reshape(NW, NCHUNK, CHUNK)
    dst2 = edge_index[1].reshape(NW, NCHUNK, CHUNK)
    h = _enc_call(x, enc_W, enc_b.reshape(1, H))
    for i in range(L):
        parts = _segsum_call(h, src2, dst2)
        h = _mlp_call(h, parts[0], parts[1], W1[i], b1[i].reshape(1, H),
                      W2[i], b2[i].reshape(1, H), gamma[i].reshape(1, H),
                      beta[i].reshape(1, H), eps[i].reshape(1, 1))
    return _pool_call(h, batch.reshape(1, N), lin_W, lin_b.reshape(1, C))


# speedup vs baseline: 15.7078x; 1.0803x over previous
import functools

import jax
import jax.numpy as jnp
from jax import lax
from jax.experimental import pallas as pl
from jax.experimental.pallas import tpu as pltpu
from jax.experimental.pallas import tpu_sc as plsc

N = 10000
E = 320000
F_IN = 128
H = 64
L = 3
C = 10
G = 64
BN_EPS = 1e-5

NC = 2
NS = 16
NW = NC * NS
CHUNK = 80
NCHUNK = E // (NW * CHUNK)
N_PAD = 10240
ROWS_PER_TILE = N_PAD // NS
ZROWS = 32
NBUF = 5


def _segsum_body(h_hbm, src_hbm, dst_hbm, out_hbm,
                 sidx, didx, gbuf, zbuf, acc, *gsem):
    c = lax.axis_index("c")
    s = lax.axis_index("s")
    wid = c * NS + s

    for r in range(ZROWS):
        for q in range(H // 16):
            zbuf[r, pl.ds(q * 16, 16)] = jnp.zeros((16,), jnp.float32)
    base = s * ROWS_PER_TILE

    def zloop(k, carry):
        pltpu.sync_copy(zbuf, acc.at[pl.ds(base + k * ZROWS, ZROWS)])
        return carry

    lax.fori_loop(0, ROWS_PER_TILE // ZROWS, zloop, 0)

    pltpu.sync_copy(src_hbm.at[wid], sidx)
    pltpu.sync_copy(dst_hbm.at[wid], didx)

    plsc.subcore_barrier()

    for b in range(NBUF):
        pltpu.async_copy(h_hbm.at[sidx.at[b]], gbuf.at[b], gsem[b])

    def group(io, carry):
        jo = io * NBUF
        for b in range(NBUF):
            j = jo + b
            pltpu.make_async_copy(h_hbm.at[sidx.at[b]],
                                  gbuf.at[b], gsem[b]).wait()
            pltpu.sync_copy(gbuf.at[b], acc.at[didx.at[j]], add=True)
            pltpu.async_copy(h_hbm.at[sidx.at[j + NBUF]], gbuf.at[b],
                             gsem[b])
        return carry

    lax.fori_loop(0, NCHUNK // NBUF - 1, group, 0)

    jo = NCHUNK - NBUF
    for b in range(NBUF):
        pltpu.make_async_copy(h_hbm.at[sidx.at[b]],
                              gbuf.at[b], gsem[b]).wait()
        pltpu.sync_copy(gbuf.at[b], acc.at[didx.at[jo + b]], add=True)

    plsc.subcore_barrier()
    pltpu.sync_copy(acc.at[pl.ds(base, ROWS_PER_TILE)],
                    out_hbm.at[c, pl.ds(base, ROWS_PER_TILE)])


_segsum_call = pl.kernel(
    _segsum_body,
    out_type=jax.ShapeDtypeStruct((NC, N_PAD, H), jnp.float32),
    mesh=plsc.VectorSubcoreMesh(core_axis_name="c", subcore_axis_name="s",
                                num_cores=NC, num_subcores=NS),
    scratch_types=[
        pltpu.VMEM((NCHUNK, CHUNK), jnp.int32),
        pltpu.VMEM((NCHUNK, CHUNK), jnp.int32),
        pltpu.VMEM((NBUF, CHUNK, H), jnp.float32),
        pltpu.VMEM((ZROWS, H), jnp.float32),
        pltpu.VMEM_SHARED((N_PAD, H), jnp.float32),
    ] + [pltpu.SemaphoreType.DMA] * NBUF,
    compiler_params=pltpu.CompilerParams(use_tc_tiling_on_sc=False),
    name="gin_segsum_sc",
)


def _enc_body(x_ref, w_ref, b_ref, out_ref):
    out_ref[:, :] = jnp.dot(x_ref[:, :], w_ref[:, :],
                            preferred_element_type=jnp.float32) + b_ref[:, :]


_enc_call = pl.pallas_call(
    _enc_body,
    out_shape=jax.ShapeDtypeStruct((N, H), jnp.float32),
    name="gin_encoder_tc",
)


def _mlp_body(h_ref, parts_ref, w1_ref, b1_ref, w2_ref, b2_ref,
              gam_ref, bet_ref, eps_ref, out_ref):
    a = ((1.0 + eps_ref[0, 0]) * h_ref[:, :]
         + parts_ref[0, :N, :] + parts_ref[1, :N, :])
    t = jnp.dot(a, w1_ref[:, :], preferred_element_type=jnp.float32)
    t = jnp.maximum(t + b1_ref[:, :], 0.0)
    h2 = jnp.dot(t, w2_ref[:, :], preferred_element_type=jnp.float32)
    h2 = h2 + b2_ref[:, :]
    mean = jnp.mean(h2, axis=0, keepdims=True)
    var = jnp.mean((h2 - mean) ** 2, axis=0, keepdims=True)
    hn = (h2 - mean) / jnp.sqrt(var + BN_EPS) * gam_ref[:, :] + bet_ref[:, :]
    out_ref[:, :] = jnp.maximum(hn, 0.0)


_mlp_call = pl.pallas_call(
    _mlp_body,
    out_shape=jax.ShapeDtypeStruct((N, H), jnp.float32),
    name="gin_layer_tc",
)


def _pool_body(h_ref, batch_ref, w_ref, b_ref, out_ref):
    gids = lax.broadcasted_iota(jnp.int32, (G, 1), 0)
    mask = (batch_ref[:, :] == gids).astype(jnp.float32)
    sums = jnp.dot(mask, h_ref[:, :], preferred_element_type=jnp.float32)
    counts = jnp.sum(mask, axis=1, keepdims=True)
    pooled = sums / jnp.maximum(counts, 1.0)
    out_ref[:, :] = jnp.dot(pooled, w_ref[:, :],
                            preferred_element_type=jnp.float32) + b_ref[:, :]


_pool_call = pl.pallas_call(
    _pool_body,
    out_shape=jax.ShapeDtypeStruct((G, C), jnp.float32),
    name="gin_pool_tc",
)


def kernel(x, edge_index, batch, enc_W, enc_b, eps, W1, b1, W2, b2,
           gamma, beta, lin_W, lin_b):
    src2 = edge_index[0].reshape(NW, NCHUNK, CHUNK)
    dst2 = edge_index[1].reshape(NW, NCHUNK, CHUNK)
    h = _enc_call(x, enc_W, enc_b.reshape(1, H))
    for i in range(L):
        parts = _segsum_call(h, src2, dst2)
        h = _mlp_call(h, parts, W1[i], b1[i].reshape(1, H),
                      W2[i], b2[i].reshape(1, H), gamma[i].reshape(1, H),
                      beta[i].reshape(1, H), eps[i].reshape(1, 1))
    return _pool_call(h, batch.reshape(1, N), lin_W, lin_b.reshape(1, C))


# speedup vs baseline: 16.3038x; 1.0379x over previous
import functools

import jax
import jax.numpy as jnp
from jax import lax
from jax.experimental import pallas as pl
from jax.experimental.pallas import tpu as pltpu
from jax.experimental.pallas import tpu_sc as plsc

N = 10000
E = 320000
F_IN = 128
H = 64
L = 3
C = 10
G = 64
BN_EPS = 1e-5

NC = 2
NS = 16
NW = NC * NS
CHUNK = 80
NCHUNK = E // (NW * CHUNK)
N_PAD = 10240
ROWS_PER_TILE = N_PAD // NS
ZROWS = 32
NBUF = 5


def _segsum_body(h_hbm, ei_hbm, out_hbm,
                 sidx, didx, gbuf, zbuf, acc, *gsem):
    c = lax.axis_index("c")
    s = lax.axis_index("s")
    wid = c * NS + s

    for r in range(ZROWS):
        for q in range(H // 16):
            zbuf[r, pl.ds(q * 16, 16)] = jnp.zeros((16,), jnp.float32)
    base = s * ROWS_PER_TILE

    def zloop(k, carry):
        pltpu.sync_copy(zbuf, acc.at[pl.ds(base + k * ZROWS, ZROWS)])
        return carry

    lax.fori_loop(0, ROWS_PER_TILE // ZROWS, zloop, 0)

    pltpu.sync_copy(ei_hbm.at[0, wid], sidx)
    pltpu.sync_copy(ei_hbm.at[1, wid], didx)

    plsc.subcore_barrier()

    for b in range(NBUF):
        pltpu.async_copy(h_hbm.at[sidx.at[b]], gbuf.at[b], gsem[b])

    def group(io, carry):
        jo = io * NBUF
        for b in range(NBUF):
            j = jo + b
            pltpu.make_async_copy(h_hbm.at[sidx.at[b]],
                                  gbuf.at[b], gsem[b]).wait()
            pltpu.sync_copy(gbuf.at[b], acc.at[didx.at[j]], add=True)
            pltpu.async_copy(h_hbm.at[sidx.at[j + NBUF]], gbuf.at[b],
                             gsem[b])
        return carry

    lax.fori_loop(0, NCHUNK // NBUF - 1, group, 0)

    jo = NCHUNK - NBUF
    for b in range(NBUF):
        pltpu.make_async_copy(h_hbm.at[sidx.at[b]],
                              gbuf.at[b], gsem[b]).wait()
        pltpu.sync_copy(gbuf.at[b], acc.at[didx.at[jo + b]], add=True)

    plsc.subcore_barrier()
    pltpu.sync_copy(acc.at[pl.ds(base, ROWS_PER_TILE)],
                    out_hbm.at[c, pl.ds(base, ROWS_PER_TILE)])


_segsum_call = pl.kernel(
    _segsum_body,
    out_type=jax.ShapeDtypeStruct((NC, N_PAD, H), jnp.float32),
    mesh=plsc.VectorSubcoreMesh(core_axis_name="c", subcore_axis_name="s",
                                num_cores=NC, num_subcores=NS),
    scratch_types=[
        pltpu.VMEM((NCHUNK, CHUNK), jnp.int32),
        pltpu.VMEM((NCHUNK, CHUNK), jnp.int32),
        pltpu.VMEM((NBUF, CHUNK, H), jnp.float32),
        pltpu.VMEM((ZROWS, H), jnp.float32),
        pltpu.VMEM_SHARED((N_PAD, H), jnp.float32),
    ] + [pltpu.SemaphoreType.DMA] * NBUF,
    compiler_params=pltpu.CompilerParams(use_tc_tiling_on_sc=False),
    name="gin_segsum_sc",
)


def _enc_body(x_ref, w_ref, b_ref, out_ref):
    out_ref[:, :] = jnp.dot(x_ref[:, :], w_ref[:, :],
                            preferred_element_type=jnp.float32) + b_ref[:, :]


_enc_call = pl.pallas_call(
    _enc_body,
    out_shape=jax.ShapeDtypeStruct((N, H), jnp.float32),
    name="gin_encoder_tc",
)


def _mlp_body(h_ref, parts_ref, w1_ref, b1_ref, w2_ref, b2_ref,
              gam_ref, bet_ref, eps_ref, out_ref):
    a = ((1.0 + eps_ref[0, 0]) * h_ref[:, :]
         + parts_ref[0, :N, :] + parts_ref[1, :N, :])
    t = jnp.dot(a, w1_ref[:, :], preferred_element_type=jnp.float32)
    t = jnp.maximum(t + b1_ref[:, :], 0.0)
    h2 = jnp.dot(t, w2_ref[:, :], preferred_element_type=jnp.float32)
    h2 = h2 + b2_ref[:, :]
    mean = jnp.mean(h2, axis=0, keepdims=True)
    var = jnp.mean((h2 - mean) ** 2, axis=0, keepdims=True)
    hn = (h2 - mean) / jnp.sqrt(var + BN_EPS) * gam_ref[:, :] + bet_ref[:, :]
    out_ref[:, :] = jnp.maximum(hn, 0.0)


_mlp_call = pl.pallas_call(
    _mlp_body,
    out_shape=jax.ShapeDtypeStruct((N, H), jnp.float32),
    name="gin_layer_tc",
)


def _pool_body(h_ref, batch_ref, w_ref, b_ref, out_ref):
    gids = lax.broadcasted_iota(jnp.int32, (G, 1), 0)
    mask = (batch_ref[:, :] == gids).astype(jnp.float32)
    sums = jnp.dot(mask, h_ref[:, :], preferred_element_type=jnp.float32)
    counts = jnp.sum(mask, axis=1, keepdims=True)
    pooled = sums / jnp.maximum(counts, 1.0)
    out_ref[:, :] = jnp.dot(pooled, w_ref[:, :],
                            preferred_element_type=jnp.float32) + b_ref[:, :]


_pool_call = pl.pallas_call(
    _pool_body,
    out_shape=jax.ShapeDtypeStruct((G, C), jnp.float32),
    name="gin_pool_tc",
)


def kernel(x, edge_index, batch, enc_W, enc_b, eps, W1, b1, W2, b2,
           gamma, beta, lin_W, lin_b):
    ei = edge_index.reshape(2, NW, NCHUNK, CHUNK)
    h = _enc_call(x, enc_W, enc_b.reshape(1, H))
    for i in range(L):
        parts = _segsum_call(h, ei)
        h = _mlp_call(h, parts, W1[i], b1[i].reshape(1, H),
                      W2[i], b2[i].reshape(1, H), gamma[i].reshape(1, H),
                      beta[i].reshape(1, H), eps[i].reshape(1, 1))
    return _pool_call(h, batch.reshape(1, N), lin_W, lin_b.reshape(1, C))
